# Initial kernel scaffold; baseline (speedup 1.0000x reference)
import jax
import jax.numpy as jnp
from jax.experimental import pallas as pl


def kernel(w_packed, shift_packed, masks, yr, yi, Pr, Pi, eig, noise_key):
    raise NotImplementedError("write your pallas kernel here")


# speedup vs baseline: 1.5873x; 1.5873x over previous
import functools

import jax
import jax.numpy as jnp
from jax.experimental import pallas as pl
from jax.experimental.pallas import tpu as pltpu

f32 = jnp.float32
bf16 = jnp.bfloat16


def _zr_kernel(div_ref, a_ref, pr_ref, pi_ref, phr_ref, phi_ref, hr_ref,
               hi_ref, yr_ref, yi_ref, zr_ref, zi_ref,
               ozr_ref, ozi_ref, rr_ref, ri_ref, eps_ref):
    Pr, Pi = pr_ref[...], pi_ref[...]
    hr, hi = hr_ref[...], hi_ref[...]
    mr = (jnp.dot(Pr, hr, preferred_element_type=f32)
          - jnp.dot(Pi, hi, preferred_element_type=f32))
    mi = (jnp.dot(Pr, hi, preferred_element_type=f32)
          + jnp.dot(Pi, hr, preferred_element_type=f32))
    d = div_ref[...]
    zr = yr_ref[...] - mr + d * zr_ref[...]
    zi = yi_ref[...] - mi + d * zi_ref[...]
    ozr_ref[...] = zr
    ozi_ref[...] = zi
    PHr, PHi = phr_ref[...], phi_ref[...]
    pr = (jnp.dot(PHr, zr, preferred_element_type=f32)
          - jnp.dot(PHi, zi, preferred_element_type=f32))
    pi = (jnp.dot(PHr, zi, preferred_element_type=f32)
          + jnp.dot(PHi, zr, preferred_element_type=f32))
    a = a_ref[...]
    rr = hr + a * pr
    ri = hi + a * pi
    rr_ref[...] = rr
    ri_ref[...] = ri
    amax = jnp.sqrt(jnp.max(rr * rr + ri * ri, axis=(0, 1), keepdims=True))
    eps_ref[...] = jnp.maximum(amax * 1e-3, 1e-5)


def _zr_step(div, inv_eig, Pr, Pi, PHr, PHi, hr, hi, yr, yi, zr, zi):
    B, M, N = Pr.shape
    T = yr.shape[-1]
    s_mn = pl.BlockSpec((None, M, N), lambda b: (b, 0, 0))
    s_nm = pl.BlockSpec((None, N, M), lambda b: (b, 0, 0))
    s_mt = pl.BlockSpec((None, M, T), lambda b: (b, 0, 0))
    s_nt = pl.BlockSpec((None, N, T), lambda b: (b, 0, 0))
    s_sc = pl.BlockSpec((None, 1, 1), lambda b: (b, 0, 0))
    return pl.pallas_call(
        _zr_kernel,
        grid=(B,),
        in_specs=[s_sc, s_sc, s_mn, s_mn, s_nm, s_nm, s_nt, s_nt,
                  s_mt, s_mt, s_mt, s_mt],
        out_specs=[s_mt, s_mt, s_nt, s_nt, s_sc],
        out_shape=[jax.ShapeDtypeStruct((B, M, T), f32),
                   jax.ShapeDtypeStruct((B, M, T), f32),
                   jax.ShapeDtypeStruct((B, N, T), f32),
                   jax.ShapeDtypeStruct((B, N, T), f32),
                   jax.ShapeDtypeStruct((B, 1, 1), f32)],
        compiler_params=pltpu.CompilerParams(dimension_semantics=("parallel",)),
    )(div, inv_eig, Pr, Pi, PHr, PHi, hr, hi, yr, yi, zr, zi)


def _dncnn_kernel(x0_ref, rd_ref, eps_ref, w0_ref, wg_ref, shift_ref,
                  masks_ref, h_ref, div_ref, *, nb, hw, n_w):
    L = 2 * hw
    offs = tuple((ky - 1) * n_w + (kx - 1) for ky in range(3) for kx in range(3))

    def tap(a16, t):
        off = offs[t]
        sh = (-off) % L
        r = a16 if sh == 0 else jnp.concatenate(
            [a16[:, L - sh:], a16[:, :L - sh]], axis=1)
        return r if off == 0 else r * masks_ref[t:t + 1, :]

    x0 = x0_ref[...]
    rd = rd_ref[...]
    eps = eps_ref[...]
    d0 = eps * rd
    a16 = jnp.concatenate([x0, d0], axis=1).astype(bf16)

    s = jnp.concatenate([tap(a16, t) for t in range(9)], axis=0)
    acc = jnp.dot(w0_ref[...], s, preferred_element_type=f32)

    for li in range(1, nb):
        pre_c = acc[:, :hw] + shift_ref[li - 1]
        pre_d = acc[:, hw:]
        cc = jnp.maximum(pre_c, 0.0)
        dd = jnp.maximum(pre_c + pre_d, 0.0) - cc
        a16 = jnp.concatenate([cc, dd], axis=1).astype(bf16)
        acc = jnp.zeros_like(acc)
        for g in range(3):
            s = jnp.concatenate([tap(a16, 3 * g + j) for j in range(3)], axis=0)
            acc = acc + jnp.dot(wg_ref[3 * (li - 1) + g], s,
                                preferred_element_type=f32)

    n_den = acc[0:2, :hw] + shift_ref[nb - 1][0:2]
    d_out = acc[0:2, hw:]
    h_ref[...] = x0 - n_den
    ssum = jnp.sum(rd * (d0 - d_out), axis=(0, 1), keepdims=True)
    div_ref[...] = ssum / (2.0 * hw) / eps


def _dncnn(x0, rd, eps, w0, wg, shift, masks16, *, nb, hw, n_w):
    B = x0.shape[0]
    kern = functools.partial(_dncnn_kernel, nb=nb, hw=hw, n_w=n_w)
    s_img = pl.BlockSpec((None, 2, hw), lambda b: (b, 0, 0))
    s_sc = pl.BlockSpec((None, 1, 1), lambda b: (b, 0, 0))
    s_w0 = pl.BlockSpec(w0.shape, lambda b: (0, 0))
    s_wg = pl.BlockSpec(wg.shape, lambda b: (0, 0, 0))
    s_sh = pl.BlockSpec(shift.shape, lambda b: (0, 0, 0))
    s_mk = pl.BlockSpec(masks16.shape, lambda b: (0, 0))
    return pl.pallas_call(
        kern,
        grid=(B,),
        in_specs=[s_img, s_img, s_sc, s_w0, s_wg, s_sh, s_mk],
        out_specs=[s_img, s_sc],
        out_shape=[jax.ShapeDtypeStruct((B, 2, hw), f32),
                   jax.ShapeDtypeStruct((B, 1, 1), f32)],
        compiler_params=pltpu.CompilerParams(dimension_semantics=("parallel",)),
    )(x0, rd, eps, w0, wg, shift, masks16)


NUM_UNROLLS = 10


@jax.jit
def kernel(w_packed, shift_packed, masks, yr, yi, Pr, Pi, eig, noise_key):
    nb = shift_packed.shape[0]
    B, M, T = yr.shape
    N = Pr.shape[-1]
    HW = N * T

    PHr = jnp.swapaxes(Pr, -1, -2)
    PHi = -jnp.swapaxes(Pi, -1, -2)
    inv_eig = (1.0 / eig).astype(f32).reshape(B, 1, 1)

    w0 = jnp.concatenate([w_packed[t][:, 0:2] for t in range(9)],
                         axis=1).astype(bf16)
    wg = jnp.stack([
        jnp.concatenate([w_packed[li * 9 + 3 * g + j] for j in range(3)],
                        axis=1)
        for li in range(1, nb) for g in range(3)], axis=0).astype(bf16)
    masks16 = masks.astype(bf16)

    nkey = jax.random.wrap_key_data(noise_key)
    rds = [jax.random.normal(jax.random.fold_in(nkey, u), (B, 2, N, T),
                             f32).reshape(B, 2, HW)
           for u in range(NUM_UNROLLS)]

    zr, zi = yr, yi
    hr = jnp.zeros((B, N, T), f32)
    hi = jnp.zeros((B, N, T), f32)
    div = jnp.zeros((B, 1, 1), f32)
    for u in range(NUM_UNROLLS):
        zr, zi, rr, ri, eps = _zr_step(div, inv_eig, Pr, Pi, PHr, PHi,
                                       hr, hi, yr, yi, zr, zi)
        x0 = jnp.stack([rr, ri], axis=1).reshape(B, 2, HW)
        h2, div = _dncnn(x0, rds[u], eps, w0, wg, shift_packed, masks16,
                         nb=nb, hw=HW, n_w=T)
        hr = h2[:, 0, :].reshape(B, N, T)
        hi = h2[:, 1, :].reshape(B, N, T)
    return hr, hi
